# baseline (speedup 1.0000x reference)
import functools

import jax
import jax.numpy as jnp
from jax import lax
from jax.experimental import pallas as pl
from jax.experimental.pallas import tpu as pltpu
from jax.experimental.pallas import tpu_sc as plsc

IN_DIM = 256
OUT_DIM = 256
N_NODES = 10000
N_EDGES = 160000

NC = 2
NS = 16
LANES = 16
NW = NC * NS

EPW = N_EDGES // NW
FULL_VREGS = EPW // LANES
TAIL = EPW - FULL_VREGS * LANES
EPW_PAD = (FULL_VREGS + 1) * LANES
N_ACC = 10240
ZERO_UNROLL = 32
SCAT_UNROLL = 13

BN = 2000
NBLK = N_NODES // BN
CHUNK = 2048
STRIP = 128
PBLOCK = NC * CHUNK

BNH = 1000
NBLKH = N_NODES // BNH

@functools.cache
def _sc_segment_sum_fn():
    mesh = plsc.VectorSubcoreMesh(core_axis_name="c", subcore_axis_name="s")

    @functools.partial(
        pl.kernel,
        out_type=jax.ShapeDtypeStruct((NBLK * PBLOCK,), jnp.float32),
        mesh=mesh,
        scratch_types=[
            pltpu.VMEM((EPW_PAD,), jnp.int32),
            pltpu.VMEM((EPW_PAD,), jnp.float32),
            pltpu.VMEM((N_ACC,), jnp.float32),
            pltpu.VMEM((NS, STRIP), jnp.float32),
            pltpu.VMEM((STRIP,), jnp.float32),
            pltpu.VMEM_SHARED((NS, N_ACC), jnp.float32),
            pltpu.SemaphoreType.DMA,
            pltpu.SemaphoreType.DMA,
        ],
        compiler_params=pltpu.CompilerParams(needs_layout_passes=False),
    )
    def _sc_segment_sum(
        idx_hbm, w_hbm, out_hbm, idx_v, w_v, acc_v, red_v, strip_v, shared,
        sem_i, sem_w,
    ):
        sid = lax.axis_index("s")
        cid = lax.axis_index("c")
        wid = sid * NC + cid
        base = wid * EPW
        cp_i = pltpu.async_copy(
            idx_hbm.at[pl.ds(base, EPW)], idx_v.at[pl.ds(0, EPW)], sem_i
        )
        cp_w = pltpu.async_copy(
            w_hbm.at[pl.ds(base, EPW)], w_v.at[pl.ds(0, EPW)], sem_w
        )

        zeros16 = jnp.zeros((LANES,), jnp.float32)

        def zero_body(i, _):
            for j in range(ZERO_UNROLL):
                acc_v[pl.ds((i * ZERO_UNROLL + j) * LANES, LANES)] = zeros16
            return 0

        lax.fori_loop(0, N_ACC // LANES // ZERO_UNROLL, zero_body, 0)
        cp_i.wait()
        cp_w.wait()

        def remap(iv):
            chunk = lax.shift_right_logical(iv * 33555, 26)
            return iv + chunk * 48

        def scat_body(i, _):
            for j in range(SCAT_UNROLL):
                off = (i * SCAT_UNROLL + j) * LANES
                iv = remap(idx_v[pl.ds(off, LANES)])
                wv = w_v[pl.ds(off, LANES)]
                plsc.addupdate_scatter(acc_v, [iv], wv)
            return 0

        lax.fori_loop(0, FULL_VREGS // SCAT_UNROLL, scat_body, 0)

        tail_off = FULL_VREGS * LANES
        iv = remap(idx_v[pl.ds(tail_off, LANES)] & 0x3FFF)
        wv = w_v[pl.ds(tail_off, LANES)]
        tmask = lax.iota(jnp.int32, LANES) < TAIL
        plsc.addupdate_scatter(acc_v, [iv], wv, mask=tmask)

        pltpu.sync_copy(acc_v, shared.at[sid])
        plsc.subcore_barrier()
        for k in range(NBLK):
            pltpu.sync_copy(
                shared.at[:, pl.ds(k * CHUNK + sid * STRIP, STRIP)], red_v
            )
            for c in range(STRIP // LANES):
                v = red_v[0, pl.ds(c * LANES, LANES)]
                for r in range(1, NS):
                    v = v + red_v[r, pl.ds(c * LANES, LANES)]
                strip_v[pl.ds(c * LANES, LANES)] = v
            pltpu.sync_copy(
                strip_v,
                out_hbm.at[pl.ds((k * NC + cid) * CHUNK + sid * STRIP, STRIP)],
            )

    return _sc_segment_sum


def _tc_hidden_body(f_ref, w_ref, b_ref, h_ref):
    h = jax.nn.relu(
        lax.dot_general(
            f_ref[...].astype(jnp.bfloat16),
            w_ref[...].astype(jnp.bfloat16),
            (((1,), (1,)), ((), ())),
            preferred_element_type=jnp.float32,
        )
        + b_ref[...]
    )
    h_ref[...] = h.astype(jnp.float8_e4m3fn)


_tc_hidden = pl.pallas_call(
    _tc_hidden_body,
    grid=(NBLKH,),
    in_specs=[
        pl.BlockSpec((BNH, IN_DIM), lambda i: (i, 0)),
        pl.BlockSpec((OUT_DIM, IN_DIM), lambda i: (0, 0)),
        pl.BlockSpec((1, OUT_DIM), lambda i: (0, 0)),
    ],
    out_specs=pl.BlockSpec((BNH, OUT_DIM), lambda i: (i, 0)),
    out_shape=jax.ShapeDtypeStruct((N_NODES, OUT_DIM), jnp.float8_e4m3fn),
)


def _tc_combine_body(p_ref, h_ref, o_ref, acc_ref, tot_ref):
    i = pl.program_id(0)

    @pl.when(i == 0)
    def _():
        acc_ref[...] = jnp.zeros_like(acc_ref)
        tot_ref[0] = 0.0

    s = p_ref[pl.ds(0, BN)]
    for r in range(1, NC):
        s = s + p_ref[pl.ds(r * CHUNK, BN)]
    contrib = lax.dot_general(
        s[None, :].astype(jnp.bfloat16), h_ref[...].astype(jnp.bfloat16),
        (((1,), (0,)), ((), ())),
        preferred_element_type=jnp.float32,
    )
    acc_ref[...] += contrib
    tot_ref[0] += jnp.sum(s)

    @pl.when(i == pl.num_programs(0) - 1)
    def _():
        o_ref[...] = acc_ref[...] / tot_ref[0]


_tc_combine = pl.pallas_call(
    _tc_combine_body,
    grid=(NBLK,),
    in_specs=[
        pl.BlockSpec((PBLOCK,), lambda i: (i,)),
        pl.BlockSpec((BN, OUT_DIM), lambda i: (i, 0)),
    ],
    out_specs=pl.BlockSpec((1, OUT_DIM), lambda i: (0, 0)),
    out_shape=jax.ShapeDtypeStruct((1, OUT_DIM), jnp.float32),
    scratch_shapes=[
        pltpu.VMEM((1, OUT_DIM), jnp.float32),
        pltpu.SMEM((1,), jnp.float32),
    ],
)


def kernel(neighbor_indices, neighbor_weights, features, W, b):
    partials = _sc_segment_sum_fn()(neighbor_indices, neighbor_weights)
    hidden = _tc_hidden(features, W, b.reshape(1, OUT_DIM))
    out = _tc_combine(partials, hidden)
    return out.reshape(OUT_DIM)


# speedup vs baseline: 1.1835x; 1.1835x over previous
import functools

import jax
import jax.numpy as jnp
from jax import lax
from jax.experimental import pallas as pl
from jax.experimental.pallas import tpu as pltpu
from jax.experimental.pallas import tpu_sc as plsc

IN_DIM = 256
OUT_DIM = 256
N_NODES = 10000
N_EDGES = 160000

NC = 2
NS = 16
LANES = 16
NW = NC * NS

EPW = N_EDGES // NW
FULL_VREGS = EPW // LANES
TAIL = EPW - FULL_VREGS * LANES
EPW_PAD = (FULL_VREGS + 1) * LANES
ZERO_UNROLL = 25
SCAT_UNROLL = 13

BN = 2000
NBLK = N_NODES // BN
CHUNK = 2048
PBLOCK = NW * CHUNK

@functools.cache
def _sc_segment_sum_fn():
    mesh = plsc.VectorSubcoreMesh(core_axis_name="c", subcore_axis_name="s")

    @functools.partial(
        pl.kernel,
        out_type=jax.ShapeDtypeStruct((NBLK * PBLOCK,), jnp.float32),
        mesh=mesh,
        scratch_types=[
            pltpu.VMEM((EPW_PAD,), jnp.int32),
            pltpu.VMEM((EPW_PAD,), jnp.float32),
            pltpu.VMEM((N_NODES,), jnp.float32),
            pltpu.SemaphoreType.DMA,
            pltpu.SemaphoreType.DMA,
        ],
        compiler_params=pltpu.CompilerParams(needs_layout_passes=False),
    )
    def _sc_segment_sum(idx_hbm, w_hbm, out_hbm, idx_v, w_v, acc_v, sem_i, sem_w):
        wid = lax.axis_index("s") * NC + lax.axis_index("c")
        base = wid * EPW
        cp_i = pltpu.async_copy(
            idx_hbm.at[pl.ds(base, EPW)], idx_v.at[pl.ds(0, EPW)], sem_i
        )
        cp_w = pltpu.async_copy(
            w_hbm.at[pl.ds(base, EPW)], w_v.at[pl.ds(0, EPW)], sem_w
        )

        zeros16 = jnp.zeros((LANES,), jnp.float32)

        def zero_body(i, _):
            for j in range(ZERO_UNROLL):
                acc_v[pl.ds((i * ZERO_UNROLL + j) * LANES, LANES)] = zeros16
            return 0

        lax.fori_loop(0, N_NODES // LANES // ZERO_UNROLL, zero_body, 0)
        cp_i.wait()
        cp_w.wait()

        def scat_body(i, _):
            for j in range(SCAT_UNROLL):
                off = (i * SCAT_UNROLL + j) * LANES
                iv = idx_v[pl.ds(off, LANES)]
                wv = w_v[pl.ds(off, LANES)]
                plsc.addupdate_scatter(acc_v, [iv], wv)
            return 0

        lax.fori_loop(0, FULL_VREGS // SCAT_UNROLL, scat_body, 0)

        tail_off = FULL_VREGS * LANES
        iv = idx_v[pl.ds(tail_off, LANES)]
        wv = w_v[pl.ds(tail_off, LANES)]
        tmask = lax.iota(jnp.int32, LANES) < TAIL
        plsc.addupdate_scatter(acc_v, [iv], wv, mask=tmask)

        for k in range(NBLK):
            pltpu.sync_copy(
                acc_v.at[pl.ds(k * BN, BN)],
                out_hbm.at[pl.ds(k * PBLOCK + wid * CHUNK, BN)],
            )

    return _sc_segment_sum


def _tc_hidden_body(f_ref, w_ref, b_ref, h_ref):
    h = jax.nn.relu(
        lax.dot_general(
            f_ref[...].astype(jnp.bfloat16),
            w_ref[...].astype(jnp.bfloat16),
            (((1,), (1,)), ((), ())),
            preferred_element_type=jnp.float32,
        )
        + b_ref[...]
    )
    h_ref[...] = h.astype(jnp.float8_e4m3fn)


_tc_hidden = pl.pallas_call(
    _tc_hidden_body,
    grid=(NBLK,),
    in_specs=[
        pl.BlockSpec((BN, IN_DIM), lambda i: (i, 0)),
        pl.BlockSpec((OUT_DIM, IN_DIM), lambda i: (0, 0)),
        pl.BlockSpec((1, OUT_DIM), lambda i: (0, 0)),
    ],
    out_specs=pl.BlockSpec((BN, OUT_DIM), lambda i: (i, 0)),
    out_shape=jax.ShapeDtypeStruct((N_NODES, OUT_DIM), jnp.float8_e4m3fn),
)


def _tc_combine_body(p_ref, h_ref, o_ref):
    acc = jnp.zeros((1, OUT_DIM), jnp.float32)
    tot = jnp.float32(0.0)
    for k in range(NBLK):
        s = p_ref[pl.ds(k * PBLOCK, BN)]
        for r in range(1, NW):
            s = s + p_ref[pl.ds(k * PBLOCK + r * CHUNK, BN)]
        acc += lax.dot_general(
            s[None, :].astype(jnp.bfloat16),
            h_ref[pl.ds(k * BN, BN), :].astype(jnp.bfloat16),
            (((1,), (0,)), ((), ())),
            preferred_element_type=jnp.float32,
        )
        tot += jnp.sum(s)
    o_ref[...] = acc / tot


_tc_combine = pl.pallas_call(
    _tc_combine_body,
    in_specs=[
        pl.BlockSpec(memory_space=pltpu.VMEM),
        pl.BlockSpec(memory_space=pltpu.VMEM),
    ],
    out_specs=pl.BlockSpec(memory_space=pltpu.VMEM),
    out_shape=jax.ShapeDtypeStruct((1, OUT_DIM), jnp.float32),
)


def kernel(neighbor_indices, neighbor_weights, features, W, b):
    partials = _sc_segment_sum_fn()(neighbor_indices, neighbor_weights)
    hidden = _tc_hidden(features, W, b.reshape(1, OUT_DIM))
    out = _tc_combine(partials, hidden)
    return out.reshape(OUT_DIM)
